# baseline (speedup 1.0000x reference)
import functools

import jax
import jax.numpy as jnp
from jax import lax
from jax.experimental import pallas as pl
from jax.experimental.pallas import tpu as pltpu
from jax.experimental.pallas import tpu_sc as plsc

NW = 32
IDXW = 128
NBUF = 3
WB = 16


def kernel(x, table):
    B, H = x.shape
    V, D = table.shape
    nb = B // NW
    nwin = nb // WB
    splits = [(c, min(IDXW, H - c)) for c in range(0, H, IDXW)]

    xi = x.astype(jnp.int32)

    mesh = plsc.VectorSubcoreMesh(core_axis_name="c", subcore_axis_name="s")

    @functools.partial(
        pl.kernel,
        mesh=mesh,
        out_type=jax.ShapeDtypeStruct((B, H, D), jnp.float32),
        scratch_types=[
            pltpu.VMEM((2, WB, H), jnp.int32),
            pltpu.VMEM((NBUF, H, D), jnp.float32),
            pltpu.SemaphoreType.DMA,
            pltpu.SemaphoreType.DMA((NBUF,)),
        ],
        compiler_params=pltpu.CompilerParams(use_tc_tiling_on_sc=False),
    )
    def gather_kernel(x_hbm, table_hbm, out_hbm, idx_v, rows_v, gsem, ssem):
        wid = lax.axis_index("s") * 2 + lax.axis_index("c")
        base_b = wid * nb

        def stage_window(w):
            pltpu.sync_copy(
                x_hbm.at[pl.ds(base_b + w * WB, WB)], idx_v.at[lax.rem(w, 2)]
            )

        def gather_descs(k):
            iw = lax.rem(k // WB, 2)
            r = lax.rem(k, WB)
            slot = lax.rem(k, NBUF)
            return [
                pltpu.make_async_copy(
                    table_hbm.at[idx_v.at[iw, r, pl.ds(c, n)]],
                    rows_v.at[slot, pl.ds(c, n)],
                    gsem,
                )
                for c, n in splits
            ]

        def store_desc(k):
            slot = lax.rem(k, NBUF)
            return pltpu.make_async_copy(
                rows_v.at[slot], out_hbm.at[base_b + k], ssem.at[slot]
            )

        stage_window(0)
        for c in gather_descs(0):
            c.start()

        def body(k, carry):
            @pl.when(k + 1 < nb)
            def _():
                @pl.when(lax.rem(k + 1, WB) == 0)
                def _():
                    stage_window((k + 1) // WB)

                @pl.when(k + 1 >= NBUF)
                def _():
                    store_desc(k + 1 - NBUF).wait()

                for c in gather_descs(k + 1):
                    c.start()

            for c in gather_descs(k):
                c.wait()
            store_desc(k).start()
            return carry

        lax.fori_loop(0, nb, body, 0)

        for t in range(NBUF):
            store_desc(nb - NBUF + t).wait()

    return gather_kernel(xi, table)


# speedup vs baseline: 1.2980x; 1.2980x over previous
import functools

import jax
import jax.numpy as jnp
from jax import lax
from jax.experimental import pallas as pl
from jax.experimental.pallas import tpu as pltpu
from jax.experimental.pallas import tpu_sc as plsc

NW = 32
IDXW = 128
CH = 256
J = CH // IDXW
NBUF = 3
IWIN = 4096
IROWS = IWIN // IDXW
CPW = IWIN // CH


def kernel(x, table):
    B, H = x.shape
    V, D = table.shape
    N = B * H
    b_per_w = N // NW
    n_ch = b_per_w // CH

    idx2d = x.reshape(N // IDXW, IDXW).astype(jnp.int32)
    wide = jnp.pad(table, ((0, 0), (0, 128 - D)))

    mesh = plsc.VectorSubcoreMesh(core_axis_name="c", subcore_axis_name="s")

    @functools.partial(
        pl.kernel,
        mesh=mesh,
        out_type=jax.ShapeDtypeStruct((N, 128), jnp.float32),
        scratch_types=[
            pltpu.VMEM((2, IROWS, IDXW), jnp.int32),
            pltpu.VMEM((NBUF, CH, 128), jnp.float32),
            pltpu.SemaphoreType.DMA,
            pltpu.SemaphoreType.DMA((NBUF,)),
        ],
    )
    def gather_kernel(idx_hbm, table_hbm, out_hbm, idx_v, rows_v, gsem, ssem):
        wid = lax.axis_index("s") * 2 + lax.axis_index("c")
        base = wid * b_per_w
        base_row = wid * (b_per_w // IDXW)

        def stage_window(w):
            row_off = pl.multiple_of(base_row + w * IROWS, 8)
            pltpu.sync_copy(idx_hbm.at[pl.ds(row_off, IROWS)], idx_v.at[lax.rem(w, 2)])

        def gather_descs(k):
            iw = lax.rem(k // CPW, 2)
            r0 = lax.rem(k, CPW) * J
            slot = lax.rem(k, NBUF)
            return [
                pltpu.make_async_copy(
                    table_hbm.at[idx_v.at[iw, r0 + j]],
                    rows_v.at[slot, pl.ds(j * IDXW, IDXW)],
                    gsem,
                )
                for j in range(J)
            ]

        def store_desc(k):
            slot = lax.rem(k, NBUF)
            return pltpu.make_async_copy(
                rows_v.at[slot],
                out_hbm.at[pl.ds(base + k * CH, CH)],
                ssem.at[slot],
            )

        stage_window(0)
        for c in gather_descs(0):
            c.start()

        def body(k, carry):
            @pl.when(k + 1 < n_ch)
            def _():
                @pl.when(lax.rem(k + 1, CPW) == 0)
                def _():
                    stage_window((k + 1) // CPW)

                @pl.when(k + 1 >= NBUF)
                def _():
                    store_desc(k + 1 - NBUF).wait()

                for c in gather_descs(k + 1):
                    c.start()

            for c in gather_descs(k):
                c.wait()
            store_desc(k).start()
            return carry

        lax.fori_loop(0, n_ch, body, 0)

        for t in range(NBUF):
            store_desc(n_ch - NBUF + t).wait()

    out128 = gather_kernel(idx2d, wide)
    return out128[:, :D].reshape(B, H, D)


# speedup vs baseline: 1.6484x; 1.2700x over previous
import functools

import jax
import jax.numpy as jnp
from jax import lax
from jax.experimental import pallas as pl
from jax.experimental.pallas import tpu as pltpu
from jax.experimental.pallas import tpu_sc as plsc

NW = 32
IDXW = 128
CH = 512
J = CH // IDXW
NBUF = 3
IWIN = 4096
IROWS = IWIN // IDXW
CPW = IWIN // CH


def kernel(x, table):
    B, H = x.shape
    V, D = table.shape
    N = B * H
    b_per_w = N // NW
    n_ch = b_per_w // CH

    idx2d = x.reshape(N // IDXW, IDXW).astype(jnp.int32)

    mesh = plsc.VectorSubcoreMesh(core_axis_name="c", subcore_axis_name="s")

    @functools.partial(
        pl.kernel,
        mesh=mesh,
        out_type=jax.ShapeDtypeStruct((N, 128), jnp.float32),
        scratch_types=[
            pltpu.VMEM((2, IROWS, IDXW), jnp.int32),
            pltpu.VMEM((NBUF, CH, D), jnp.float32),
            pltpu.SemaphoreType.DMA,
            pltpu.SemaphoreType.DMA((NBUF,)),
        ],
        compiler_params=pltpu.CompilerParams(use_tc_tiling_on_sc=False),
    )
    def gather_kernel(idx_hbm, table_hbm, out_hbm, idx_v, rows_v, gsem, ssem):
        wid = lax.axis_index("s") * 2 + lax.axis_index("c")
        base = wid * b_per_w
        base_row = wid * (b_per_w // IDXW)

        def stage_window(w):
            row_off = pl.multiple_of(base_row + w * IROWS, 8)
            pltpu.sync_copy(idx_hbm.at[pl.ds(row_off, IROWS)], idx_v.at[lax.rem(w, 2)])

        def gather_descs(k):
            iw = lax.rem(k // CPW, 2)
            r0 = lax.rem(k, CPW) * J
            slot = lax.rem(k, NBUF)
            return [
                pltpu.make_async_copy(
                    table_hbm.at[idx_v.at[iw, r0 + j]],
                    rows_v.at[slot, pl.ds(j * IDXW, IDXW)],
                    gsem,
                )
                for j in range(J)
            ]

        def store_desc(k):
            slot = lax.rem(k, NBUF)
            return pltpu.make_async_copy(
                rows_v.at[slot],
                out_hbm.at[pl.ds(base + k * CH, CH), pl.ds(0, D)],
                ssem.at[slot],
            )

        stage_window(0)
        for c in gather_descs(0):
            c.start()

        def body(k, carry):
            @pl.when(k + 1 < n_ch)
            def _():
                @pl.when(lax.rem(k + 1, CPW) == 0)
                def _():
                    stage_window((k + 1) // CPW)

                @pl.when(k + 1 >= NBUF)
                def _():
                    store_desc(k + 1 - NBUF).wait()

                for c in gather_descs(k + 1):
                    c.start()

            for c in gather_descs(k):
                c.wait()
            store_desc(k).start()
            return carry

        lax.fori_loop(0, n_ch, body, 0)

        for t in range(NBUF):
            store_desc(n_ch - NBUF + t).wait()

    out128 = gather_kernel(idx2d, table)
    return out128[:, :D].reshape(B, H, D)


# speedup vs baseline: 1.6489x; 1.0003x over previous
import functools

import jax
import jax.numpy as jnp
from jax import lax
from jax.experimental import pallas as pl
from jax.experimental.pallas import tpu as pltpu
from jax.experimental.pallas import tpu_sc as plsc

NW = 32
IDXW = 128
CH = 512
J = CH // IDXW
NBUF = 3
IWIN = 4096
IROWS = IWIN // IDXW
CPW = IWIN // CH


def kernel(x, table):
    B, H = x.shape
    V, D = table.shape
    N = B * H
    b_per_w = N // NW
    n_ch = b_per_w // CH

    idx2d = x.reshape(N // IDXW, IDXW).astype(jnp.int32)

    mesh = plsc.VectorSubcoreMesh(core_axis_name="c", subcore_axis_name="s")

    @functools.partial(
        pl.kernel,
        mesh=mesh,
        out_type=jax.ShapeDtypeStruct((N, 128), jnp.float32),
        scratch_types=[
            pltpu.VMEM((2, IROWS, IDXW), jnp.int32),
            pltpu.VMEM((NBUF, CH, D), jnp.float32),
            pltpu.SemaphoreType.DMA,
            pltpu.SemaphoreType.DMA((NBUF,)),
            pltpu.SemaphoreType.DMA((2,)),
        ],
        compiler_params=pltpu.CompilerParams(use_tc_tiling_on_sc=False),
    )
    def gather_kernel(idx_hbm, table_hbm, out_hbm, idx_v, rows_v, gsem, ssem, isem):
        wid = lax.axis_index("s") * 2 + lax.axis_index("c")
        base = wid * b_per_w
        base_row = wid * (b_per_w // IDXW)
        n_win = n_ch // CPW

        def stage_desc(w):
            row_off = pl.multiple_of(base_row + w * IROWS, 8)
            sl = lax.rem(w, 2)
            return pltpu.make_async_copy(
                idx_hbm.at[pl.ds(row_off, IROWS)], idx_v.at[sl], isem.at[sl]
            )

        def gather_descs(k):
            iw = lax.rem(k // CPW, 2)
            r0 = lax.rem(k, CPW) * J
            slot = lax.rem(k, NBUF)
            return [
                pltpu.make_async_copy(
                    table_hbm.at[idx_v.at[iw, r0 + j]],
                    rows_v.at[slot, pl.ds(j * IDXW, IDXW)],
                    gsem,
                )
                for j in range(J)
            ]

        def store_desc(k):
            slot = lax.rem(k, NBUF)
            return pltpu.make_async_copy(
                rows_v.at[slot],
                out_hbm.at[pl.ds(base + k * CH, CH), pl.ds(0, D)],
                ssem.at[slot],
            )

        d0 = stage_desc(0)
        d0.start()
        d0.wait()
        stage_desc(1).start()
        for c in gather_descs(0):
            c.start()

        def body(k, carry):
            @pl.when(k + 1 < n_ch)
            def _():
                @pl.when(lax.rem(k + 1, CPW) == 0)
                def _():
                    stage_desc((k + 1) // CPW).wait()

                @pl.when(k + 1 >= NBUF)
                def _():
                    store_desc(k + 1 - NBUF).wait()

                for c in gather_descs(k + 1):
                    c.start()

            for c in gather_descs(k):
                c.wait()
            store_desc(k).start()

            @pl.when((lax.rem(k + 1, CPW) == 0) & (k + 1 < n_ch))
            def _():
                w2 = (k + 1) // CPW + 1

                @pl.when(w2 < n_win)
                def _():
                    stage_desc(w2).start()

            return carry

        lax.fori_loop(0, n_ch, body, 0)

        for t in range(NBUF):
            store_desc(n_ch - NBUF + t).wait()

    out128 = gather_kernel(idx2d, table)
    return out128[:, :D].reshape(B, H, D)


# speedup vs baseline: 1.6528x; 1.0024x over previous
import functools

import jax
import jax.numpy as jnp
from jax import lax
from jax.experimental import pallas as pl
from jax.experimental.pallas import tpu as pltpu
from jax.experimental.pallas import tpu_sc as plsc

NW = 32
IDXW = 128
CH = 512
J = CH // IDXW
NBUF = 3
IWIN = 4096
IROWS = IWIN // IDXW
CPW = IWIN // CH


def kernel(x, table):
    B, H = x.shape
    V, D = table.shape
    N = B * H
    b_per_w = N // NW
    n_ch = b_per_w // CH

    idx2d = x.reshape(N // IDXW, IDXW).astype(jnp.int32)

    mesh = plsc.VectorSubcoreMesh(core_axis_name="c", subcore_axis_name="s")

    @functools.partial(
        pl.kernel,
        mesh=mesh,
        out_type=jax.ShapeDtypeStruct((N, 128), jnp.float32),
        scratch_types=[
            pltpu.VMEM((2, IROWS, IDXW), jnp.int32),
            pltpu.VMEM((NBUF, CH, D), jnp.float32),
            pltpu.SemaphoreType.DMA,
            pltpu.SemaphoreType.DMA((NBUF,)),
        ],
        compiler_params=pltpu.CompilerParams(use_tc_tiling_on_sc=False),
    )
    def gather_kernel(idx_hbm, table_hbm, out_hbm, idx_v, rows_v, gsem, ssem):
        wid = lax.axis_index("s") * 2 + lax.axis_index("c")
        base = wid * b_per_w
        base_row = wid * (b_per_w // IDXW)

        def stage_window(w):
            row_off = pl.multiple_of(base_row + w * IROWS, 8)
            pltpu.sync_copy(idx_hbm.at[pl.ds(row_off, IROWS)], idx_v.at[lax.rem(w, 2)])

        def gather_descs(k):
            iw = lax.rem(k // CPW, 2)
            r0 = lax.rem(k, CPW) * J
            slot = lax.rem(k, NBUF)
            return [
                pltpu.make_async_copy(
                    table_hbm.at[idx_v.at[iw, r0 + j]],
                    rows_v.at[slot, pl.ds(j * IDXW, IDXW)],
                    gsem,
                )
                for j in range(J)
            ]

        def store_desc(k):
            slot = lax.rem(k, NBUF)
            return pltpu.make_async_copy(
                rows_v.at[slot],
                out_hbm.at[pl.ds(base + k * CH, CH), pl.ds(0, D)],
                ssem.at[slot],
            )

        stage_window(0)
        for c in gather_descs(0):
            c.start()

        def body(k, carry):
            @pl.when(k + 1 < n_ch)
            def _():
                @pl.when(lax.rem(k + 1, CPW) == 0)
                def _():
                    stage_window((k + 1) // CPW)

                @pl.when(k + 1 >= NBUF)
                def _():
                    store_desc(k + 1 - NBUF).wait()

                for c in gather_descs(k + 1):
                    c.start()

            for c in gather_descs(k):
                c.wait()
            store_desc(k).start()
            return carry

        lax.fori_loop(0, n_ch, body, 0)

        for t in range(NBUF):
            store_desc(n_ch - NBUF + t).wait()

    out128 = gather_kernel(idx2d, table)
    return out128[:, :D].reshape(B, H, D)
